# baseline (speedup 1.0000x reference)
import functools

import jax
import jax.numpy as jnp
from jax import lax
from jax.experimental import pallas as pl
from jax.experimental.pallas import tpu as pltpu
from jax.experimental.pallas import tpu_sc as plsc

CUTOFF = 5.0


def _mlp_body(a_ref, w1_ref, b1_ref, w2_ref, b2_ref, out_ref):
    a = a_ref[...]
    h = lax.dot_general(a, w1_ref[...], (((1,), (1,)), ((), ())),
                        preferred_element_type=jnp.float32) + b1_ref[...]
    h = h * jax.nn.sigmoid(h)
    out_ref[...] = lax.dot_general(h, w2_ref[...], (((1,), (1,)), ((), ())),
                                   preferred_element_type=jnp.float32) + b2_ref[...]


def _mlp(a2, W1, b1, W2, b2):
    M, F = a2.shape
    return pl.pallas_call(
        _mlp_body,
        out_shape=jax.ShapeDtypeStruct((M, F), jnp.float32),
    )(a2, W1, b1.reshape(1, F), W2, b2.reshape(1, F))


def _make_gather(b, B, A, NN, F):
    info = plsc.get_sparse_core_info()
    NC, NS = info.num_cores, info.num_subcores
    NW = NC * NS
    Eb = A * NN
    per_w = Eb // NW
    CH = 128
    n_chunks = per_w // CH
    gbase0 = b * Eb
    b_add = b * A
    L = 16

    mesh = plsc.VectorSubcoreMesh(core_axis_name="c", subcore_axis_name="s")

    @functools.partial(
        pl.kernel,
        mesh=mesh,
        out_type=jax.ShapeDtypeStruct((Eb, F), jnp.float32),
        scratch_types=[
            pltpu.VMEM((CH,), jnp.int32),
            pltpu.VMEM((CH,), jnp.int32),
            pltpu.VMEM((CH, F), jnp.float32),
            pltpu.VMEM((CH, F), jnp.float32),
            pltpu.SemaphoreType.DMA,
            pltpu.SemaphoreType.DMA,
        ],
    )
    def gather_k(table_hbm, idx_hbm, out_hbm,
                 idx_v0, idx_v1, rows_v0, rows_v1, sem0, sem1):
        wid = lax.axis_index("s") * NC + lax.axis_index("c")
        base = wid * per_w

        idx_bufs = (idx_v0, idx_v1)
        row_bufs = (rows_v0, rows_v1)
        sems = (sem0, sem1)

        def load_and_fire(ci, slot):
            cbase = base + ci * CH
            idx_v, rows_v, sem = idx_bufs[slot], row_bufs[slot], sems[slot]
            pltpu.sync_copy(idx_hbm.at[pl.ds(gbase0 + cbase, CH)], idx_v)
            for m in range(CH // L):
                sl = pl.ds(m * L, L)
                idx_v[sl] = idx_v[sl] + b_add
            pltpu.async_copy(table_hbm.at[idx_v], rows_v, sem)

        def drain(ci, slot):
            cbase = base + ci * CH
            rows_v, sem = row_bufs[slot], sems[slot]
            pltpu.make_async_copy(table_hbm.at[idx_bufs[slot]], rows_v, sem).wait()
            pltpu.sync_copy(rows_v, out_hbm.at[pl.ds(cbase, CH)])

        load_and_fire(0, 0)

        def body(ci, _):
            slot = lax.rem(ci, 2)

            @pl.when(ci + 1 < n_chunks)
            def _():
                lax.switch(1 - slot, [lambda: load_and_fire(ci + 1, 0),
                                      lambda: load_and_fire(ci + 1, 1)])

            lax.switch(slot, [lambda: drain(ci, 0), lambda: drain(ci, 1)])
            return 0

        lax.fori_loop(0, n_chunks, body, 0)

    return gather_k


def _fuse_body(p_ref, aj_ref, rbf_ref, d_ref, nm_ref, a_ref, am_ref,
               wr_ref, br_ref, *rest, RB, NN, F):
    aout_ref, pout_ref = rest[-2], rest[-1]
    rbfm = lax.dot_general(rbf_ref[...], wr_ref[...], (((1,), (0,)), ((), ())),
                           preferred_element_type=jnp.float32) + br_ref[...]
    d = d_ref[0]
    x = d * (1.0 / CUTOFF)
    x2 = x * x
    x4 = x2 * x2
    x9 = x4 * x4 * x
    f = 1.0 + x9 * (-55.0 + x * (99.0 - 45.0 * x))
    cut = jnp.where(d < CUTOFF, f, 0.0)
    scale = jnp.swapaxes(cut * nm_ref[0], 0, 1)

    rep = jnp.broadcast_to(scale[:, None, :], (RB, NN, NN)).reshape(RB * NN, NN)
    j_lane = lax.broadcasted_iota(jnp.int32, (RB * NN, NN), 1)
    j_row = lax.broadcasted_iota(jnp.int32, (RB * NN, NN), 0) % NN
    masked = jnp.where(j_lane == j_row, rep, 0.0)
    scale_e = lax.dot_general(masked, jnp.ones((NN, F), jnp.float32),
                              (((1,), (0,)), ((), ())),
                              preferred_element_type=jnp.float32)

    am = am_ref[...]
    ai = jnp.broadcast_to(am[:, None, :], (RB, NN, F)).reshape(RB * NN, F)
    msij = ai * aj_ref[...] * rbfm * scale_e
    pout_ref[...] = p_ref[...] + msij
    aout_ref[...] = a_ref[...] + msij.reshape(RB, NN, F).sum(axis=1)


def _fuse_chunk(b, p_f, aj_b, rbf_b, d_v, nm_v, a2, amsij, W_rbfT, b_rbf,
                aprev, pprev, RB):
    M, F = a2.shape
    E = p_f.shape[0]
    R = rbf_b.shape[1]
    NN = E // M
    A = M // d_v.shape[0]
    steps = A // RB
    eb = RB * NN
    body = functools.partial(_fuse_body, RB=RB, NN=NN, F=F)
    in_specs = [
        pl.BlockSpec((eb, F), lambda g: (b * steps + g, 0)),
        pl.BlockSpec((eb, F), lambda g: (g, 0)),
        pl.BlockSpec((eb, R), lambda g: (b * steps + g, 0)),
        pl.BlockSpec((1, NN, RB), lambda g: (b, 0, g)),
        pl.BlockSpec((1, NN, RB), lambda g: (b, 0, g)),
        pl.BlockSpec((RB, F), lambda g: (b * steps + g, 0)),
        pl.BlockSpec((RB, F), lambda g: (b * steps + g, 0)),
        pl.BlockSpec((R, F), lambda g: (0, 0)),
        pl.BlockSpec((1, F), lambda g: (0, 0)),
    ]
    args = [p_f, aj_b, rbf_b, d_v, nm_v, a2, amsij, W_rbfT,
            b_rbf.reshape(1, F)]
    aliases = {}
    if aprev is not None:
        in_specs += [pl.BlockSpec(memory_space=pl.ANY),
                     pl.BlockSpec(memory_space=pl.ANY)]
        args += [aprev, pprev]
        aliases = {9: 0, 10: 1}
    return pl.pallas_call(
        body,
        grid=(steps,),
        in_specs=in_specs,
        out_specs=[
            pl.BlockSpec((RB, F), lambda g: (b * steps + g, 0)),
            pl.BlockSpec((eb, F), lambda g: (b * steps + g, 0)),
        ],
        out_shape=[
            jax.ShapeDtypeStruct((M, F), jnp.float32),
            jax.ShapeDtypeStruct((E, F), jnp.float32),
        ],
        input_output_aliases=aliases,
    )(*args)


def kernel(a, p, rbf, D, N, NM, W_rbf, b_rbf, W1, b1, W2, b2):
    B, A, NN, F = p.shape
    R = rbf.shape[-1]
    M = B * A
    E = M * NN

    a2 = a.reshape(M, F)
    amsij = _mlp(a2, W1, b1, W2, b2)

    d_v = D.transpose(0, 2, 1)
    nm_v = NM.transpose(0, 2, 1)
    w_rbf_t = W_rbf.transpose(1, 0)

    idx = N.reshape(E)
    ajs = [_make_gather(b, B, A, NN, F)(amsij, idx) for b in range(B)]

    p_f = p.reshape(E, F)
    rbf_f = rbf.reshape(E, R)
    aout, pout = None, None
    for b in range(B):
        aout, pout = _fuse_chunk(b, p_f, ajs[b], rbf_f, d_v, nm_v, a2,
                                 amsij, w_rbf_t, b_rbf, aout, pout, RB=128)
    return aout.reshape(B, A, F), pout.reshape(B, A, NN, F)


# speedup vs baseline: 1.0917x; 1.0917x over previous
import functools

import jax
import jax.numpy as jnp
from jax import lax
from jax.experimental import pallas as pl
from jax.experimental.pallas import tpu as pltpu
from jax.experimental.pallas import tpu_sc as plsc

CUTOFF = 5.0


def _mlp_body(a_ref, w1_ref, b1_ref, w2_ref, b2_ref, out_ref):
    a = a_ref[...]
    h = lax.dot_general(a, w1_ref[...], (((1,), (1,)), ((), ())),
                        preferred_element_type=jnp.float32) + b1_ref[...]
    h = h * jax.nn.sigmoid(h)
    out_ref[...] = lax.dot_general(h, w2_ref[...], (((1,), (1,)), ((), ())),
                                   preferred_element_type=jnp.float32) + b2_ref[...]


def _mlp(a2, W1, b1, W2, b2):
    M, F = a2.shape
    return pl.pallas_call(
        _mlp_body,
        out_shape=jax.ShapeDtypeStruct((M, F), jnp.float32),
    )(a2, W1, b1.reshape(1, F), W2, b2.reshape(1, F))


def _make_gather(h, CB, A, NN, F):
    info = plsc.get_sparse_core_info()
    NC, NS = info.num_cores, info.num_subcores
    NW = NC * NS
    Eb = CB * A * NN
    per_w = Eb // NW
    CH = 128
    n_chunks = per_w // CH
    w_per_b = NW // CB
    gbase0 = h * Eb
    L = 16

    mesh = plsc.VectorSubcoreMesh(core_axis_name="c", subcore_axis_name="s")

    @functools.partial(
        pl.kernel,
        mesh=mesh,
        out_type=jax.ShapeDtypeStruct((Eb, F), jnp.float32),
        scratch_types=[
            pltpu.VMEM((CH,), jnp.int32),
            pltpu.VMEM((CH,), jnp.int32),
            pltpu.VMEM((CH, F), jnp.float32),
            pltpu.VMEM((CH, F), jnp.float32),
            pltpu.SemaphoreType.DMA,
            pltpu.SemaphoreType.DMA,
        ],
    )
    def gather_k(table_hbm, idx_hbm, out_hbm,
                 idx_v0, idx_v1, rows_v0, rows_v1, sem0, sem1):
        wid = lax.axis_index("s") * NC + lax.axis_index("c")
        base = wid * per_w
        b_add = (h * CB + wid // w_per_b) * A

        idx_bufs = (idx_v0, idx_v1)
        row_bufs = (rows_v0, rows_v1)
        sems = (sem0, sem1)

        def load_and_fire(ci, slot):
            cbase = base + ci * CH
            idx_v, rows_v, sem = idx_bufs[slot], row_bufs[slot], sems[slot]
            pltpu.sync_copy(idx_hbm.at[pl.ds(gbase0 + cbase, CH)], idx_v)
            for m in range(CH // L):
                sl = pl.ds(m * L, L)
                idx_v[sl] = idx_v[sl] + b_add
            pltpu.async_copy(table_hbm.at[idx_v], rows_v, sem)

        def drain(ci, slot):
            cbase = base + ci * CH
            rows_v, sem = row_bufs[slot], sems[slot]
            pltpu.make_async_copy(table_hbm.at[idx_bufs[slot]], rows_v, sem).wait()
            pltpu.sync_copy(rows_v, out_hbm.at[pl.ds(cbase, CH)])

        load_and_fire(0, 0)

        def body(ci, _):
            slot = lax.rem(ci, 2)

            @pl.when(ci + 1 < n_chunks)
            def _():
                lax.switch(1 - slot, [lambda: load_and_fire(ci + 1, 0),
                                      lambda: load_and_fire(ci + 1, 1)])

            lax.switch(slot, [lambda: drain(ci, 0), lambda: drain(ci, 1)])
            return 0

        lax.fori_loop(0, n_chunks, body, 0)

    return gather_k


def _fuse_body(p_ref, aj_ref, rbf_ref, d_ref, nm_ref, a_ref, am_ref,
               wr_ref, br_ref, *rest, RB, NN, F):
    aout_ref, pout_ref = rest[-2], rest[-1]
    rbfm = lax.dot_general(rbf_ref[...], wr_ref[...], (((1,), (0,)), ((), ())),
                           preferred_element_type=jnp.float32) + br_ref[...]
    d = d_ref[0]
    x = d * (1.0 / CUTOFF)
    x2 = x * x
    x4 = x2 * x2
    x9 = x4 * x4 * x
    f = 1.0 + x9 * (-55.0 + x * (99.0 - 45.0 * x))
    cut = jnp.where(d < CUTOFF, f, 0.0)
    scale = jnp.swapaxes(cut * nm_ref[0], 0, 1)

    rep = jnp.broadcast_to(scale[:, None, :], (RB, NN, NN)).reshape(RB * NN, NN)
    j_lane = lax.broadcasted_iota(jnp.int32, (RB * NN, NN), 1)
    j_row = lax.broadcasted_iota(jnp.int32, (RB * NN, NN), 0) % NN
    masked = jnp.where(j_lane == j_row, rep, 0.0)
    scale_e = lax.dot_general(masked, jnp.ones((NN, F), jnp.float32),
                              (((1,), (0,)), ((), ())),
                              preferred_element_type=jnp.float32)

    am = am_ref[...]
    ai = jnp.broadcast_to(am[:, None, :], (RB, NN, F)).reshape(RB * NN, F)
    msij = ai * aj_ref[...] * rbfm * scale_e
    pout_ref[...] = p_ref[...] + msij
    aout_ref[...] = a_ref[...] + msij.reshape(RB, NN, F).sum(axis=1)


def _fuse_chunk(h, CB, p_f, aj_h, rbf_f, d_v, nm_v, a2, amsij, W_rbfT, b_rbf,
                aprev, pprev, RB):
    M, F = a2.shape
    E = p_f.shape[0]
    R = rbf_f.shape[1]
    NN = E // M
    A = M // d_v.shape[0]
    spb = A // RB
    steps = CB * spb
    eb = RB * NN
    body = functools.partial(_fuse_body, RB=RB, NN=NN, F=F)
    in_specs = [
        pl.BlockSpec((eb, F), lambda g: (h * steps + g, 0)),
        pl.BlockSpec((eb, F), lambda g: (g, 0)),
        pl.BlockSpec((eb, R), lambda g: (h * steps + g, 0)),
        pl.BlockSpec((1, NN, RB),
                     lambda g: (h * CB + g // spb, 0, g % spb)),
        pl.BlockSpec((1, NN, RB),
                     lambda g: (h * CB + g // spb, 0, g % spb)),
        pl.BlockSpec((RB, F), lambda g: (h * steps + g, 0)),
        pl.BlockSpec((RB, F), lambda g: (h * steps + g, 0)),
        pl.BlockSpec((R, F), lambda g: (0, 0)),
        pl.BlockSpec((1, F), lambda g: (0, 0)),
    ]
    args = [p_f, aj_h, rbf_f, d_v, nm_v, a2, amsij, W_rbfT,
            b_rbf.reshape(1, F)]
    aliases = {}
    if aprev is not None:
        in_specs += [pl.BlockSpec(memory_space=pl.ANY),
                     pl.BlockSpec(memory_space=pl.ANY)]
        args += [aprev, pprev]
        aliases = {9: 0, 10: 1}
    return pl.pallas_call(
        body,
        grid=(steps,),
        in_specs=in_specs,
        out_specs=[
            pl.BlockSpec((RB, F), lambda g: (h * steps + g, 0)),
            pl.BlockSpec((eb, F), lambda g: (h * steps + g, 0)),
        ],
        out_shape=[
            jax.ShapeDtypeStruct((M, F), jnp.float32),
            jax.ShapeDtypeStruct((E, F), jnp.float32),
        ],
        input_output_aliases=aliases,
    )(*args)


def kernel(a, p, rbf, D, N, NM, W_rbf, b_rbf, W1, b1, W2, b2):
    B, A, NN, F = p.shape
    R = rbf.shape[-1]
    M = B * A
    E = M * NN

    a2 = a.reshape(M, F)
    amsij = _mlp(a2, W1, b1, W2, b2)

    d_v = D.transpose(0, 2, 1)
    nm_v = NM.transpose(0, 2, 1)
    w_rbf_t = W_rbf.transpose(1, 0)

    CHUNKS = 2
    CB = B // CHUNKS
    idx = N.reshape(E)
    ajs = [_make_gather(h, CB, A, NN, F)(amsij, idx) for h in range(CHUNKS)]

    p_f = p.reshape(E, F)
    rbf_f = rbf.reshape(E, R)
    aout, pout = None, None
    for h in range(CHUNKS):
        aout, pout = _fuse_chunk(h, CB, p_f, ajs[h], rbf_f, d_v, nm_v, a2,
                                 amsij, w_rbf_t, b_rbf, aout, pout, RB=128)
    return aout.reshape(B, A, F), pout.reshape(B, A, NN, F)


# speedup vs baseline: 1.1504x; 1.0538x over previous
import functools

import jax
import jax.numpy as jnp
from jax import lax
from jax.experimental import pallas as pl
from jax.experimental.pallas import tpu as pltpu
from jax.experimental.pallas import tpu_sc as plsc

CUTOFF = 5.0


def _mlp_body(a_ref, w1_ref, b1_ref, w2_ref, b2_ref, out_ref):
    a = a_ref[...]
    h = lax.dot_general(a, w1_ref[...], (((1,), (1,)), ((), ())),
                        preferred_element_type=jnp.float32) + b1_ref[...]
    h = h * jax.nn.sigmoid(h)
    out_ref[...] = lax.dot_general(h, w2_ref[...], (((1,), (1,)), ((), ())),
                                   preferred_element_type=jnp.float32) + b2_ref[...]


def _mlp(a2, W1, b1, W2, b2):
    M, F = a2.shape
    return pl.pallas_call(
        _mlp_body,
        out_shape=jax.ShapeDtypeStruct((M, F), jnp.float32),
    )(a2, W1, b1.reshape(1, F), W2, b2.reshape(1, F))


def _make_gather(h, CB, A, NN, F):
    info = plsc.get_sparse_core_info()
    NC, NS = info.num_cores, info.num_subcores
    NW = NC * NS
    Eb = CB * A * NN
    per_w = Eb // NW
    CH = 128
    n_chunks = per_w // CH
    w_per_b = NW // CB
    gbase0 = h * Eb
    L = 16

    mesh = plsc.VectorSubcoreMesh(core_axis_name="c", subcore_axis_name="s")

    @functools.partial(
        pl.kernel,
        mesh=mesh,
        out_type=jax.ShapeDtypeStruct((Eb, F), jnp.float32),
        scratch_types=[
            pltpu.VMEM((CH,), jnp.int32),
            pltpu.VMEM((CH,), jnp.int32),
            pltpu.VMEM((CH, F), jnp.float32),
            pltpu.VMEM((CH, F), jnp.float32),
            pltpu.SemaphoreType.DMA,
            pltpu.SemaphoreType.DMA,
        ],
    )
    def gather_k(table_hbm, idx_hbm, out_hbm,
                 idx_v0, idx_v1, rows_v0, rows_v1, sem0, sem1):
        wid = lax.axis_index("s") * NC + lax.axis_index("c")
        base = wid * per_w
        b_add = (h * CB + wid // w_per_b) * A

        idx_bufs = (idx_v0, idx_v1)
        row_bufs = (rows_v0, rows_v1)
        sems = (sem0, sem1)

        def load_and_fire(ci, slot):
            cbase = base + ci * CH
            idx_v, rows_v, sem = idx_bufs[slot], row_bufs[slot], sems[slot]
            pltpu.sync_copy(idx_hbm.at[pl.ds(gbase0 + cbase, CH)], idx_v)
            for m in range(CH // L):
                sl = pl.ds(m * L, L)
                idx_v[sl] = idx_v[sl] + b_add
            pltpu.async_copy(table_hbm.at[idx_v], rows_v, sem)

        def drain(ci, slot):
            cbase = base + ci * CH
            rows_v, sem = row_bufs[slot], sems[slot]
            pltpu.make_async_copy(table_hbm.at[idx_bufs[slot]], rows_v, sem).wait()
            pltpu.sync_copy(rows_v, out_hbm.at[pl.ds(cbase, CH)])

        load_and_fire(0, 0)

        def body(ci, _):
            slot = lax.rem(ci, 2)

            @pl.when(ci + 1 < n_chunks)
            def _():
                lax.switch(1 - slot, [lambda: load_and_fire(ci + 1, 0),
                                      lambda: load_and_fire(ci + 1, 1)])

            lax.switch(slot, [lambda: drain(ci, 0), lambda: drain(ci, 1)])
            return 0

        lax.fori_loop(0, n_chunks, body, 0)

    return gather_k


def _fuse_body(p_ref, aj_ref, rbf_ref, d_ref, nm_ref, a_ref, am_ref,
               wr_ref, br_ref, *rest, RB, NN, F):
    aout_ref, pout_ref = rest[-2], rest[-1]
    rbfm = lax.dot_general(rbf_ref[...], wr_ref[...], (((1,), (0,)), ((), ())),
                           preferred_element_type=jnp.float32) + br_ref[...]
    d = d_ref[0]
    x = d * (1.0 / CUTOFF)
    x2 = x * x
    x4 = x2 * x2
    x9 = x4 * x4 * x
    f = 1.0 + x9 * (-55.0 + x * (99.0 - 45.0 * x))
    cut = jnp.where(d < CUTOFF, f, 0.0)
    scale = jnp.swapaxes(cut * nm_ref[0], 0, 1)

    rep = jnp.broadcast_to(scale[:, None, :], (RB, NN, NN)).reshape(RB * NN, NN)
    j_lane = lax.broadcasted_iota(jnp.int32, (RB * NN, NN), 1)
    j_row = lax.broadcasted_iota(jnp.int32, (RB * NN, NN), 0) % NN
    masked = jnp.where(j_lane == j_row, rep, 0.0)
    scale_e = lax.dot_general(masked, jnp.ones((NN, F), jnp.float32),
                              (((1,), (0,)), ((), ())),
                              preferred_element_type=jnp.float32)

    am = am_ref[...]
    ai = jnp.broadcast_to(am[:, None, :], (RB, NN, F)).reshape(RB * NN, F)
    msij = ai * aj_ref[...] * rbfm * scale_e
    pout_ref[...] = p_ref[...] + msij
    aout_ref[...] = a_ref[...] + msij.reshape(RB, NN, F).sum(axis=1)


def _fuse_chunk(h, CB, p_f, aj_h, rbf_f, d_v, nm_v, a2, amsij, W_rbfT, b_rbf,
                aprev, pprev, RB):
    M, F = a2.shape
    E = p_f.shape[0]
    R = rbf_f.shape[1]
    NN = E // M
    A = M // d_v.shape[0]
    spb = A // RB
    steps = CB * spb
    eb = RB * NN
    body = functools.partial(_fuse_body, RB=RB, NN=NN, F=F)
    in_specs = [
        pl.BlockSpec((eb, F), lambda g: (h * steps + g, 0)),
        pl.BlockSpec((eb, F), lambda g: (g, 0)),
        pl.BlockSpec((eb, R), lambda g: (h * steps + g, 0)),
        pl.BlockSpec((1, NN, RB),
                     lambda g: (h * CB + g // spb, 0, g % spb)),
        pl.BlockSpec((1, NN, RB),
                     lambda g: (h * CB + g // spb, 0, g % spb)),
        pl.BlockSpec((RB, F), lambda g: (h * steps + g, 0)),
        pl.BlockSpec((RB, F), lambda g: (h * steps + g, 0)),
        pl.BlockSpec((R, F), lambda g: (0, 0)),
        pl.BlockSpec((1, F), lambda g: (0, 0)),
    ]
    args = [p_f, aj_h, rbf_f, d_v, nm_v, a2, amsij, W_rbfT,
            b_rbf.reshape(1, F)]
    aliases = {}
    if aprev is not None:
        in_specs += [pl.BlockSpec(memory_space=pl.ANY),
                     pl.BlockSpec(memory_space=pl.ANY)]
        args += [aprev, pprev]
        aliases = {9: 0, 10: 1}
    return pl.pallas_call(
        body,
        grid=(steps,),
        in_specs=in_specs,
        out_specs=[
            pl.BlockSpec((RB, F), lambda g: (h * steps + g, 0)),
            pl.BlockSpec((eb, F), lambda g: (h * steps + g, 0)),
        ],
        out_shape=[
            jax.ShapeDtypeStruct((M, F), jnp.float32),
            jax.ShapeDtypeStruct((E, F), jnp.float32),
        ],
        input_output_aliases=aliases,
    )(*args)


def kernel(a, p, rbf, D, N, NM, W_rbf, b_rbf, W1, b1, W2, b2):
    B, A, NN, F = p.shape
    R = rbf.shape[-1]
    M = B * A
    E = M * NN

    a2 = a.reshape(M, F)
    amsij = _mlp(a2, W1, b1, W2, b2)

    d_v = D.transpose(0, 2, 1)
    nm_v = NM.transpose(0, 2, 1)
    w_rbf_t = W_rbf.transpose(1, 0)

    CHUNKS = 1
    CB = B // CHUNKS
    idx = N.reshape(E)
    ajs = [_make_gather(h, CB, A, NN, F)(amsij, idx) for h in range(CHUNKS)]

    p_f = p.reshape(E, F)
    rbf_f = rbf.reshape(E, R)
    aout, pout = None, None
    for h in range(CHUNKS):
        aout, pout = _fuse_chunk(h, CB, p_f, ajs[h], rbf_f, d_v, nm_v, a2,
                                 amsij, w_rbf_t, b_rbf, aout, pout, RB=128)
    return aout.reshape(B, A, F), pout.reshape(B, A, NN, F)


# speedup vs baseline: 1.2940x; 1.1248x over previous
import functools

import jax
import jax.numpy as jnp
from jax import lax
from jax.experimental import pallas as pl
from jax.experimental.pallas import tpu as pltpu
from jax.experimental.pallas import tpu_sc as plsc

CUTOFF = 5.0


def _mlp_body(a_ref, w1_ref, b1_ref, w2_ref, b2_ref, out_ref):
    a = a_ref[...]
    h = lax.dot_general(a, w1_ref[...], (((1,), (1,)), ((), ())),
                        preferred_element_type=jnp.float32) + b1_ref[...]
    h = h * jax.nn.sigmoid(h)
    out_ref[...] = lax.dot_general(h, w2_ref[...], (((1,), (1,)), ((), ())),
                                   preferred_element_type=jnp.float32) + b2_ref[...]


def _mlp(a2, W1, b1, W2, b2):
    M, F = a2.shape
    return pl.pallas_call(
        _mlp_body,
        out_shape=jax.ShapeDtypeStruct((M, F), jnp.float32),
    )(a2, W1, b1.reshape(1, F), W2, b2.reshape(1, F))


def _make_gather(h, CB, A, NN, F):
    info = plsc.get_sparse_core_info()
    NC, NS = info.num_cores, info.num_subcores
    NW = NC * NS
    Eb = CB * A * NN
    per_w = Eb // NW
    CH = 128
    n_chunks = per_w // CH
    w_per_b = NW // CB
    gbase0 = h * Eb
    L = 16

    mesh = plsc.VectorSubcoreMesh(core_axis_name="c", subcore_axis_name="s")

    @functools.partial(
        pl.kernel,
        mesh=mesh,
        out_type=jax.ShapeDtypeStruct((Eb, F), jnp.float32),
        scratch_types=[
            pltpu.VMEM((CH,), jnp.int32),
            pltpu.VMEM((CH,), jnp.int32),
            pltpu.VMEM((CH, F), jnp.float32),
            pltpu.VMEM((CH, F), jnp.float32),
            pltpu.SemaphoreType.DMA,
            pltpu.SemaphoreType.DMA,
        ],
    )
    def gather_k(table_hbm, idx_hbm, out_hbm,
                 idx_v0, idx_v1, rows_v0, rows_v1, sem0, sem1):
        wid = lax.axis_index("s") * NC + lax.axis_index("c")
        base = wid * per_w
        b_add = (h * CB + wid // w_per_b) * A

        idx_bufs = (idx_v0, idx_v1)
        row_bufs = (rows_v0, rows_v1)
        sems = (sem0, sem1)

        def load_and_fire(ci, slot):
            cbase = base + ci * CH
            idx_v, rows_v, sem = idx_bufs[slot], row_bufs[slot], sems[slot]
            pltpu.sync_copy(idx_hbm.at[pl.ds(gbase0 + cbase, CH)], idx_v)
            for m in range(CH // L):
                sl = pl.ds(m * L, L)
                idx_v[sl] = idx_v[sl] + b_add
            pltpu.async_copy(table_hbm.at[idx_v], rows_v, sem)

        def drain(ci, slot):
            cbase = base + ci * CH
            rows_v, sem = row_bufs[slot], sems[slot]
            pltpu.make_async_copy(table_hbm.at[idx_bufs[slot]], rows_v, sem).wait()
            pltpu.sync_copy(rows_v, out_hbm.at[pl.ds(cbase, CH)])

        load_and_fire(0, 0)

        def body(ci, _):
            slot = lax.rem(ci, 2)

            @pl.when(ci + 1 < n_chunks)
            def _():
                lax.switch(1 - slot, [lambda: load_and_fire(ci + 1, 0),
                                      lambda: load_and_fire(ci + 1, 1)])

            lax.switch(slot, [lambda: drain(ci, 0), lambda: drain(ci, 1)])
            return 0

        lax.fori_loop(0, n_chunks, body, 0)

    return gather_k


def _fuse_body(p_ref, aj_ref, rbf_ref, d_ref, nm_ref, a_ref, am_ref,
               wr_ref, br_ref, *rest, RB, NN, F, R):
    aout_ref, pout_ref, rbfm_s = rest[-3], rest[-2], rest[-1]

    rblk = rbf_ref[0].reshape(NN * R, RB)
    rt = jnp.swapaxes(rblk, 0, 1)
    w = wr_ref[...]
    br = br_ref[...]
    for j in range(NN):
        rbfm_s[:, j, :] = lax.dot_general(
            rt[:, j * R:(j + 1) * R], w, (((1,), (0,)), ((), ())),
            preferred_element_type=jnp.float32) + br
    rbfm = rbfm_s[...].reshape(RB * NN, F)
    d = d_ref[0]
    x = d * (1.0 / CUTOFF)
    x2 = x * x
    x4 = x2 * x2
    x9 = x4 * x4 * x
    f = 1.0 + x9 * (-55.0 + x * (99.0 - 45.0 * x))
    cut = jnp.where(d < CUTOFF, f, 0.0)
    scale = jnp.swapaxes(cut * nm_ref[0], 0, 1)

    rep = jnp.broadcast_to(scale[:, None, :], (RB, NN, NN)).reshape(RB * NN, NN)
    j_lane = lax.broadcasted_iota(jnp.int32, (RB * NN, NN), 1)
    j_row = lax.broadcasted_iota(jnp.int32, (RB * NN, NN), 0) % NN
    masked = jnp.where(j_lane == j_row, rep, 0.0)
    scale_e = lax.dot_general(masked, jnp.ones((NN, F), jnp.float32),
                              (((1,), (0,)), ((), ())),
                              preferred_element_type=jnp.float32)

    am = am_ref[...]
    ai = jnp.broadcast_to(am[:, None, :], (RB, NN, F)).reshape(RB * NN, F)
    msij = ai * aj_ref[...] * rbfm * scale_e
    pout_ref[...] = p_ref[...] + msij
    aout_ref[...] = a_ref[...] + msij.reshape(RB, NN, F).sum(axis=1)


def _fuse_chunk(h, CB, p_f, aj_h, rbf_v, d_v, nm_v, a2, amsij, W_rbfT, b_rbf,
                aprev, pprev, RB):
    M, F = a2.shape
    E = p_f.shape[0]
    R = rbf_v.shape[2]
    NN = E // M
    A = M // d_v.shape[0]
    spb = A // RB
    steps = CB * spb
    eb = RB * NN
    body = functools.partial(_fuse_body, RB=RB, NN=NN, F=F, R=R)
    in_specs = [
        pl.BlockSpec((eb, F), lambda g: (h * steps + g, 0)),
        pl.BlockSpec((eb, F), lambda g: (g, 0)),
        pl.BlockSpec((1, NN, R, RB),
                     lambda g: (h * CB + g // spb, 0, 0, g % spb)),
        pl.BlockSpec((1, NN, RB),
                     lambda g: (h * CB + g // spb, 0, g % spb)),
        pl.BlockSpec((1, NN, RB),
                     lambda g: (h * CB + g // spb, 0, g % spb)),
        pl.BlockSpec((RB, F), lambda g: (h * steps + g, 0)),
        pl.BlockSpec((RB, F), lambda g: (h * steps + g, 0)),
        pl.BlockSpec((R, F), lambda g: (0, 0)),
        pl.BlockSpec((1, F), lambda g: (0, 0)),
    ]
    args = [p_f, aj_h, rbf_v, d_v, nm_v, a2, amsij, W_rbfT,
            b_rbf.reshape(1, F)]
    aliases = {}
    if aprev is not None:
        in_specs += [pl.BlockSpec(memory_space=pl.ANY),
                     pl.BlockSpec(memory_space=pl.ANY)]
        args += [aprev, pprev]
        aliases = {9: 0, 10: 1}
    return pl.pallas_call(
        body,
        grid=(steps,),
        in_specs=in_specs,
        out_specs=[
            pl.BlockSpec((RB, F), lambda g: (h * steps + g, 0)),
            pl.BlockSpec((eb, F), lambda g: (h * steps + g, 0)),
        ],
        out_shape=[
            jax.ShapeDtypeStruct((M, F), jnp.float32),
            jax.ShapeDtypeStruct((E, F), jnp.float32),
        ],
        scratch_shapes=[pltpu.VMEM((RB, NN, F), jnp.float32)],
        input_output_aliases=aliases,
    )(*args)


def kernel(a, p, rbf, D, N, NM, W_rbf, b_rbf, W1, b1, W2, b2):
    B, A, NN, F = p.shape
    R = rbf.shape[-1]
    M = B * A
    E = M * NN

    a2 = a.reshape(M, F)
    amsij = _mlp(a2, W1, b1, W2, b2)

    d_v = D.transpose(0, 2, 1)
    nm_v = NM.transpose(0, 2, 1)
    w_rbf_t = W_rbf.transpose(1, 0)

    CHUNKS = 1
    CB = B // CHUNKS
    idx = N.reshape(E)
    ajs = [_make_gather(h, CB, A, NN, F)(amsij, idx) for h in range(CHUNKS)]

    p_f = p.reshape(E, F)
    rbf_v = rbf.transpose(0, 2, 3, 1)
    aout, pout = None, None
    for h in range(CHUNKS):
        aout, pout = _fuse_chunk(h, CB, p_f, ajs[h], rbf_v, d_v, nm_v, a2,
                                 amsij, w_rbf_t, b_rbf, aout, pout, RB=128)
    return aout.reshape(B, A, F), pout.reshape(B, A, NN, F)
